# Initial kernel scaffold; baseline (speedup 1.0000x reference)
import jax
import jax.numpy as jnp
from jax.experimental import pallas as pl


def kernel(x, edge_index, edge_attr, batch, x_emb1, x_emb2, edge_emb1, edge_emb2, mlp_w1, mlp_b1, mlp_w2, mlp_b2, bn_gamma, bn_beta, feat_w, feat_b, pred_w1, pred_b1, pred_w2, pred_b2, pred_w3, pred_b3):
    raise NotImplementedError("write your pallas kernel here")


# speedup vs baseline: 1.7929x; 1.7929x over previous
import functools

import jax
import jax.numpy as jnp
from jax.experimental import pallas as pl
from jax.experimental.pallas import tpu as pltpu

EK = 512
SPAN = EK + 8
NBLK = 512


def _scatter_kernel(lo_ref, loc_ref, g_ref, out_ref, *, npad):
    i = pl.program_id(0)

    @pl.when(i == 0)
    def _init():
        out_ref[...] = jnp.zeros((npad, 64), jnp.float32)

    loc = loc_ref[...]
    cols = jax.lax.broadcasted_iota(jnp.int32, (EK, SPAN), 1)
    onehot = (cols == loc).astype(jnp.float32)
    contrib = jax.lax.dot_general(
        onehot, g_ref[...], (((0,), (0,)), ((), ())),
        preferred_element_type=jnp.float32, precision=jax.lax.Precision.HIGHEST)
    lo = lo_ref[i]
    cur = out_ref[pl.ds(lo, SPAN), :]
    out_ref[pl.ds(lo, SPAN), :] = cur + contrib


def _dense_kernel(aggr_ref, c1_ref, c2_ref, e1_ref, e2_ref,
                  w1_ref, b1_ref, w2_ref, b2_ref,
                  hpre_ref, stats_ref, *, n):
    i = pl.program_id(0)
    aggr = aggr_ref[...]
    aggr = aggr + jnp.dot(c1_ref[...], e1_ref[...],
                          preferred_element_type=jnp.float32, precision=jax.lax.Precision.HIGHEST)
    aggr = aggr + jnp.dot(c2_ref[...], e2_ref[...],
                          preferred_element_type=jnp.float32, precision=jax.lax.Precision.HIGHEST)
    hm = jnp.maximum(
        jnp.dot(aggr, w1_ref[...], preferred_element_type=jnp.float32, precision=jax.lax.Precision.HIGHEST)
        + b1_ref[...], 0.0)
    hpre = (jnp.dot(hm, w2_ref[...], preferred_element_type=jnp.float32, precision=jax.lax.Precision.HIGHEST)
            + b2_ref[...])
    hpre_ref[...] = hpre

    @pl.when(i == 0)
    def _init():
        stats_ref[...] = jnp.zeros((8, 64), jnp.float32)

    gidx = jax.lax.broadcasted_iota(jnp.int32, (NBLK, 1), 0) + i * NBLK
    mask = (gidx < n).astype(jnp.float32)
    hm_ = hpre * mask
    s = jnp.sum(hm_, axis=0, keepdims=True)
    sq = jnp.sum(hm_ * hpre, axis=0, keepdims=True)
    stats_ref[0:1, :] = stats_ref[0:1, :] + s
    stats_ref[1:2, :] = stats_ref[1:2, :] + sq


def _apply_kernel(hpre_ref, stats_ref, gam_ref, bet_ref, out_ref,
                  *, n, relu):
    mean = stats_ref[0:1, :] / n
    var = stats_ref[1:2, :] / n - mean * mean
    h = ((hpre_ref[...] - mean) * jax.lax.rsqrt(var + 1e-5)
         * gam_ref[...] + bet_ref[...])
    if relu:
        h = jnp.maximum(h, 0.0)
    out_ref[...] = h


def _head_kernel(h_ref, b_ref, fw_ref, fb_ref, pw1_ref, pb1_ref,
                 pw2_ref, pb2_ref, pw3_ref, pb3_ref,
                 num_ref, cnt_ref, z_ref, *, nsteps):
    i = pl.program_id(0)

    @pl.when(i == 0)
    def _init():
        num_ref[...] = jnp.zeros((256, 64), jnp.float32)
        cnt_ref[...] = jnp.zeros((256, 64), jnp.float32)

    b = b_ref[...]
    cols = jax.lax.broadcasted_iota(jnp.int32, (NBLK, 256), 1)
    onehot = (cols == b).astype(jnp.float32)
    num_ref[...] = num_ref[...] + jax.lax.dot_general(
        onehot, h_ref[...], (((0,), (0,)), ((), ())),
        preferred_element_type=jnp.float32, precision=jax.lax.Precision.HIGHEST)
    cnt_ref[...] = cnt_ref[...] + jax.lax.dot_general(
        onehot, jnp.ones((NBLK, 64), jnp.float32), (((0,), (0,)), ((), ())),
        preferred_element_type=jnp.float32, precision=jax.lax.Precision.HIGHEST)

    @pl.when(i == nsteps - 1)
    def _finish():
        hg = num_ref[...] / jnp.maximum(cnt_ref[...], 1.0)
        hf = (jnp.dot(hg, fw_ref[...], preferred_element_type=jnp.float32, precision=jax.lax.Precision.HIGHEST)
              + fb_ref[...])
        z = jax.nn.softplus(
            jnp.dot(hf, pw1_ref[...], preferred_element_type=jnp.float32, precision=jax.lax.Precision.HIGHEST)
            + pb1_ref[...])
        z = jax.nn.softplus(
            jnp.dot(z, pw2_ref[...], preferred_element_type=jnp.float32, precision=jax.lax.Precision.HIGHEST)
            + pb2_ref[...])
        z_ref[...] = (jnp.dot(z, pw3_ref[...],
                              preferred_element_type=jnp.float32, precision=jax.lax.Precision.HIGHEST)
                      + pb3_ref[...])


def kernel(x, edge_index, edge_attr, batch, x_emb1, x_emb2, edge_emb1,
           edge_emb2, mlp_w1, mlp_b1, mlp_w2, mlp_b2, bn_gamma, bn_beta,
           feat_w, feat_b, pred_w1, pred_b1, pred_w2, pred_b2, pred_w3,
           pred_b3):
    n = x.shape[0]
    e = edge_index.shape[1]
    nlayer = edge_emb1.shape[0]
    f32 = jnp.float32

    h = x_emb1[x[:, 0]] + x_emb2[x[:, 1]]

    loop = jnp.arange(n, dtype=edge_index.dtype)
    src = jnp.concatenate([edge_index[0], loop]).astype(jnp.int32)
    dst = jnp.concatenate([edge_index[1], loop]).astype(jnp.int32)
    ea0 = jnp.concatenate(
        [edge_attr[:, 0], jnp.full((n,), 4, edge_attr.dtype)]).astype(jnp.int32)
    ea1 = jnp.concatenate(
        [edge_attr[:, 1], jnp.zeros((n,), edge_attr.dtype)]).astype(jnp.int32)

    perm = jnp.argsort(dst)
    dst_s = dst[perm]
    src_s = src[perm]

    etot = e + n
    nebl = -(-etot // EK)
    epad = nebl * EK

    lo8 = (dst_s[jnp.arange(nebl) * EK] // 8) * 8
    loc = dst_s - jnp.repeat(lo8, EK)[:etot]
    loc = jnp.concatenate(
        [loc, jnp.full((epad - etot,), SPAN + 7, jnp.int32)])
    loc = loc.reshape(epad, 1)
    src_sp = jnp.concatenate(
        [src_s, jnp.zeros((epad - etot,), jnp.int32)])

    c1 = jax.ops.segment_sum(jax.nn.one_hot(ea0, 8, dtype=f32),
                             dst, num_segments=n)
    c2 = jax.ops.segment_sum(jax.nn.one_hot(ea1, 8, dtype=f32),
                             dst, num_segments=n)

    nnbl = -(-n // NBLK)
    npad_n = nnbl * NBLK
    padn = npad_n - n
    c1 = jnp.pad(c1, ((0, padn), (0, 0)))
    c2 = jnp.pad(c2, ((0, padn), (0, 0)))
    npad_sc = ((n // 8) * 8 + SPAN + 8)

    scatter = pl.pallas_call(
        functools.partial(_scatter_kernel, npad=npad_sc),
        grid_spec=pltpu.PrefetchScalarGridSpec(
            num_scalar_prefetch=1,
            grid=(nebl,),
            in_specs=[
                pl.BlockSpec((EK, 1), lambda i, lo: (i, 0)),
                pl.BlockSpec((EK, 64), lambda i, lo: (i, 0)),
            ],
            out_specs=pl.BlockSpec((npad_sc, 64), lambda i, lo: (0, 0)),
        ),
        out_shape=jax.ShapeDtypeStruct((npad_sc, 64), f32),
    )

    dense = pl.pallas_call(
        functools.partial(_dense_kernel, n=n),
        grid=(nnbl,),
        in_specs=[
            pl.BlockSpec((NBLK, 64), lambda i: (i, 0)),
            pl.BlockSpec((NBLK, 8), lambda i: (i, 0)),
            pl.BlockSpec((NBLK, 8), lambda i: (i, 0)),
            pl.BlockSpec((8, 64), lambda i: (0, 0)),
            pl.BlockSpec((8, 64), lambda i: (0, 0)),
            pl.BlockSpec((64, 128), lambda i: (0, 0)),
            pl.BlockSpec((1, 128), lambda i: (0, 0)),
            pl.BlockSpec((128, 64), lambda i: (0, 0)),
            pl.BlockSpec((1, 64), lambda i: (0, 0)),
        ],
        out_specs=[
            pl.BlockSpec((NBLK, 64), lambda i: (i, 0)),
            pl.BlockSpec((8, 64), lambda i: (0, 0)),
        ],
        out_shape=[
            jax.ShapeDtypeStruct((npad_n, 64), f32),
            jax.ShapeDtypeStruct((8, 64), f32),
        ],
    )

    def make_apply(relu):
        return pl.pallas_call(
            functools.partial(_apply_kernel, n=n, relu=relu),
            grid=(nnbl,),
            in_specs=[
                pl.BlockSpec((NBLK, 64), lambda i: (i, 0)),
                pl.BlockSpec((8, 64), lambda i: (0, 0)),
                pl.BlockSpec((1, 64), lambda i: (0, 0)),
                pl.BlockSpec((1, 64), lambda i: (0, 0)),
            ],
            out_specs=pl.BlockSpec((NBLK, 64), lambda i: (i, 0)),
            out_shape=jax.ShapeDtypeStruct((npad_n, 64), f32),
        )

    apply_relu = make_apply(True)
    apply_last = make_apply(False)

    e1p = jnp.pad(edge_emb1, ((0, 0), (0, 3), (0, 0)))
    e2p = jnp.pad(edge_emb2, ((0, 0), (0, 5), (0, 0)))

    for l in range(nlayer):
        g = jnp.take(h, src_sp, axis=0)
        aggr = scatter(lo8, loc, g)
        hpre, stats = dense(aggr[:npad_n], c1, c2, e1p[l], e2p[l],
                            mlp_w1[l], mlp_b1[l].reshape(1, -1),
                            mlp_w2[l], mlp_b2[l].reshape(1, -1))
        app = apply_relu if l < nlayer - 1 else apply_last
        hp = app(hpre, stats, bn_gamma[l].reshape(1, -1),
                 bn_beta[l].reshape(1, -1))
        h = hp[:n]

    bpad = jnp.concatenate(
        [batch.astype(jnp.int32), jnp.full((padn,), 256, jnp.int32)])
    bpad = bpad.reshape(npad_n, 1)
    hpad = jnp.pad(h, ((0, padn), (0, 0)))
    pw3p = jnp.pad(pred_w3, ((0, 0), (0, 7)))
    pb3p = jnp.pad(pred_b3, ((0, 7))).reshape(1, 8)

    head = pl.pallas_call(
        functools.partial(_head_kernel, nsteps=nnbl),
        grid=(nnbl,),
        in_specs=[
            pl.BlockSpec((NBLK, 64), lambda i: (i, 0)),
            pl.BlockSpec((NBLK, 1), lambda i: (i, 0)),
            pl.BlockSpec((64, 512), lambda i: (0, 0)),
            pl.BlockSpec((1, 512), lambda i: (0, 0)),
            pl.BlockSpec((512, 256), lambda i: (0, 0)),
            pl.BlockSpec((1, 256), lambda i: (0, 0)),
            pl.BlockSpec((256, 256), lambda i: (0, 0)),
            pl.BlockSpec((1, 256), lambda i: (0, 0)),
            pl.BlockSpec((256, 8), lambda i: (0, 0)),
            pl.BlockSpec((1, 8), lambda i: (0, 0)),
        ],
        out_specs=[
            pl.BlockSpec((256, 64), lambda i: (0, 0)),
            pl.BlockSpec((256, 64), lambda i: (0, 0)),
            pl.BlockSpec((256, 8), lambda i: (0, 0)),
        ],
        out_shape=[
            jax.ShapeDtypeStruct((256, 64), f32),
            jax.ShapeDtypeStruct((256, 64), f32),
            jax.ShapeDtypeStruct((256, 8), f32),
        ],
    )
    _, _, z = head(hpad, bpad, feat_w, feat_b.reshape(1, -1),
                   pred_w1, pred_b1.reshape(1, -1),
                   pred_w2, pred_b2.reshape(1, -1),
                   pw3p, pb3p)
    return z[:, :1]
